# Initial kernel scaffold; baseline (speedup 1.0000x reference)
import jax
import jax.numpy as jnp
from jax.experimental import pallas as pl


def kernel(x, hop_feat, idx, W1, b1, W2, b2):
    raise NotImplementedError("write your pallas kernel here")


# speedup vs baseline: 3.3743x; 3.3743x over previous
import functools
import math

import jax
import jax.numpy as jnp
from jax import lax
from jax.experimental import pallas as pl
from jax.experimental.pallas import tpu as pltpu
from jax.experimental.pallas import tpu_sc as plsc

_NFEAT = 128
_NCLASS = 16
_B = 100000
_NC = 2
_NS = 16
_NW = _NC * _NS
_CHUNK = 128
_CPW = 25
_B_PAD = _NW * _CPW * _CHUNK


def _sc_gather(x, idx2d):
    mesh = plsc.VectorSubcoreMesh(core_axis_name="c", subcore_axis_name="s")

    @functools.partial(
        pl.kernel,
        out_type=jax.ShapeDtypeStruct((_B_PAD, _NFEAT), jnp.float32),
        mesh=mesh,
        scratch_types=[
            pltpu.VMEM((_CPW + 7, _CHUNK), jnp.int32),
            pltpu.VMEM((_CHUNK, _NFEAT), jnp.float32),
            pltpu.SemaphoreType.DMA,
        ],
    )
    def gather_kernel(x_hbm, idx_hbm, out_hbm, idx_v, rows_v, sem):
        wid = lax.axis_index("s") * _NC + lax.axis_index("c")
        row0 = wid * _CPW
        start = pl.multiple_of(row0 // 8 * 8, 8)
        off = row0 - start
        pltpu.sync_copy(idx_hbm.at[pl.ds(start, _CPW + 7)], idx_v)

        def body(j, carry):
            pltpu.async_copy(x_hbm.at[idx_v.at[off + j]], rows_v, sem).wait()
            pltpu.sync_copy(rows_v, out_hbm.at[pl.ds((row0 + j) * _CHUNK, _CHUNK)])
            return carry

        lax.fori_loop(0, _CPW, body, 0)

    return gather_kernel(x, idx2d)


def _tc_fused(anchor, hop_feat, W1, b1, W2, b2, bm):
    scale = 1.0 / math.sqrt(float(_NFEAT))

    def body(a_ref, h_ref, w1_ref, b1_ref, w2_ref, b2_ref, o_ref):
        a = a_ref[...]
        norm = jnp.sqrt(jnp.sum(a * a, axis=1, keepdims=True))
        an = a / jnp.maximum(norm, 1e-12)
        h0 = h_ref[0]
        h1 = h_ref[1]
        h2 = h_ref[2]
        l0 = jnp.sum(an * an, axis=1, keepdims=True) * scale
        l1 = jnp.sum(an * h0, axis=1, keepdims=True) * scale
        l2 = jnp.sum(an * h1, axis=1, keepdims=True) * scale
        l3 = jnp.sum(an * h2, axis=1, keepdims=True) * scale
        m = jnp.maximum(jnp.maximum(l0, l1), jnp.maximum(l2, l3))
        e0 = jnp.exp(l0 - m)
        e1 = jnp.exp(l1 - m)
        e2 = jnp.exp(l2 - m)
        e3 = jnp.exp(l3 - m)
        inv = 1.0 / (e0 + e1 + e2 + e3)
        pooled = (e0 * an + e1 * h0 + e2 * h1 + e3 * h2) * inv
        h = jnp.dot(pooled, w1_ref[...], preferred_element_type=jnp.float32)
        h = jnp.maximum(h + b1_ref[...], 0.0)
        o = jnp.dot(h, w2_ref[...], preferred_element_type=jnp.float32) + b2_ref[...]
        om = jnp.max(o, axis=1, keepdims=True)
        o_ref[...] = (o - om) - jnp.log(
            jnp.sum(jnp.exp(o - om), axis=1, keepdims=True))

    return pl.pallas_call(
        body,
        grid=(_B // bm,),
        in_specs=[
            pl.BlockSpec((bm, _NFEAT), lambda i: (i, 0)),
            pl.BlockSpec((3, bm, _NFEAT), lambda i: (0, i, 0)),
            pl.BlockSpec((_NFEAT, _NFEAT), lambda i: (0, 0)),
            pl.BlockSpec((1, _NFEAT), lambda i: (0, 0)),
            pl.BlockSpec((_NFEAT, _NCLASS), lambda i: (0, 0)),
            pl.BlockSpec((1, _NCLASS), lambda i: (0, 0)),
        ],
        out_specs=pl.BlockSpec((bm, _NCLASS), lambda i: (i, 0)),
        out_shape=jax.ShapeDtypeStruct((_B, _NCLASS), jnp.float32),
    )(anchor, hop_feat, W1, b1, W2, b2)


def kernel(x, hop_feat, idx, W1, b1, W2, b2):
    idx32 = idx.astype(jnp.int32)
    idx_pad = jnp.concatenate(
        [idx32, jnp.zeros((_B_PAD - _B,), jnp.int32)])
    idx2d = idx_pad.reshape(_NW * _CPW, _CHUNK)
    anchor = _sc_gather(x, idx2d)
    return _tc_fused(anchor, hop_feat, W1, b1.reshape(1, _NFEAT),
                     W2, b2.reshape(1, _NCLASS), bm=2000)


# speedup vs baseline: 3.5681x; 1.0574x over previous
import functools
import math

import jax
import jax.numpy as jnp
from jax import lax
from jax.experimental import pallas as pl
from jax.experimental.pallas import tpu as pltpu
from jax.experimental.pallas import tpu_sc as plsc

_NFEAT = 128
_NCLASS = 16
_B = 100000
_NC = 2
_NS = 16
_NW = _NC * _NS
_CHUNK = 128
_CPW = 25
_B_PAD = _NW * _CPW * _CHUNK


def _sc_gather(x, idx2d):
    mesh = plsc.VectorSubcoreMesh(core_axis_name="c", subcore_axis_name="s")

    @functools.partial(
        pl.kernel,
        out_type=jax.ShapeDtypeStruct((_B_PAD, _NFEAT), jnp.float32),
        mesh=mesh,
        scratch_types=[
            pltpu.VMEM((_CPW + 7, _CHUNK), jnp.int32),
            pltpu.VMEM((2, _CHUNK, _NFEAT), jnp.float32),
            pltpu.SemaphoreType.DMA((2,)),
        ],
    )
    def gather_kernel(x_hbm, idx_hbm, out_hbm, idx_v, rows_v, sems):
        wid = lax.axis_index("s") * _NC + lax.axis_index("c")
        row0 = wid * _CPW
        start = pl.multiple_of(row0 // 8 * 8, 8)
        off = row0 - start
        pltpu.sync_copy(idx_hbm.at[pl.ds(start, _CPW + 7)], idx_v)

        pltpu.async_copy(x_hbm.at[idx_v.at[off]], rows_v.at[0], sems.at[0])

        def body(j, carry):
            slot = lax.rem(j, 2)
            nslot = lax.rem(j + 1, 2)

            @pl.when(j + 1 < _CPW)
            def _():
                pltpu.async_copy(x_hbm.at[idx_v.at[off + j + 1]],
                                 rows_v.at[nslot], sems.at[nslot])

            pltpu.make_async_copy(x_hbm.at[idx_v.at[off + j]],
                                  rows_v.at[slot], sems.at[slot]).wait()
            pltpu.sync_copy(rows_v.at[slot],
                            out_hbm.at[pl.ds((row0 + j) * _CHUNK, _CHUNK)])
            return carry

        lax.fori_loop(0, _CPW, body, 0)

    return gather_kernel(x, idx2d)


def _tc_fused(anchor, hop_feat, W1, b1, W2, b2, bm):
    scale = 1.0 / math.sqrt(float(_NFEAT))

    def body(a_ref, h_ref, w1_ref, b1_ref, w2_ref, b2_ref, o_ref):
        a = a_ref[...]
        norm = jnp.sqrt(jnp.sum(a * a, axis=1, keepdims=True))
        an = a / jnp.maximum(norm, 1e-12)
        h0 = h_ref[0]
        h1 = h_ref[1]
        h2 = h_ref[2]
        l0 = jnp.sum(an * an, axis=1, keepdims=True) * scale
        l1 = jnp.sum(an * h0, axis=1, keepdims=True) * scale
        l2 = jnp.sum(an * h1, axis=1, keepdims=True) * scale
        l3 = jnp.sum(an * h2, axis=1, keepdims=True) * scale
        m = jnp.maximum(jnp.maximum(l0, l1), jnp.maximum(l2, l3))
        e0 = jnp.exp(l0 - m)
        e1 = jnp.exp(l1 - m)
        e2 = jnp.exp(l2 - m)
        e3 = jnp.exp(l3 - m)
        inv = 1.0 / (e0 + e1 + e2 + e3)
        pooled = (e0 * an + e1 * h0 + e2 * h1 + e3 * h2) * inv
        h = jnp.dot(pooled, w1_ref[...], preferred_element_type=jnp.float32)
        h = jnp.maximum(h + b1_ref[...], 0.0)
        o = jnp.dot(h, w2_ref[...], preferred_element_type=jnp.float32) + b2_ref[...]
        om = jnp.max(o, axis=1, keepdims=True)
        o_ref[...] = (o - om) - jnp.log(
            jnp.sum(jnp.exp(o - om), axis=1, keepdims=True))

    return pl.pallas_call(
        body,
        grid=(_B // bm,),
        in_specs=[
            pl.BlockSpec((bm, _NFEAT), lambda i: (i, 0)),
            pl.BlockSpec((3, bm, _NFEAT), lambda i: (0, i, 0)),
            pl.BlockSpec((_NFEAT, _NFEAT), lambda i: (0, 0)),
            pl.BlockSpec((1, _NFEAT), lambda i: (0, 0)),
            pl.BlockSpec((_NFEAT, _NCLASS), lambda i: (0, 0)),
            pl.BlockSpec((1, _NCLASS), lambda i: (0, 0)),
        ],
        out_specs=pl.BlockSpec((bm, _NCLASS), lambda i: (i, 0)),
        out_shape=jax.ShapeDtypeStruct((_B, _NCLASS), jnp.float32),
    )(anchor, hop_feat, W1, b1, W2, b2)


def kernel(x, hop_feat, idx, W1, b1, W2, b2):
    idx32 = idx.astype(jnp.int32)
    idx_pad = jnp.concatenate(
        [idx32, jnp.zeros((_B_PAD - _B,), jnp.int32)])
    idx2d = idx_pad.reshape(_NW * _CPW, _CHUNK)
    anchor = _sc_gather(x, idx2d)
    return _tc_fused(anchor, hop_feat, W1, b1.reshape(1, _NFEAT),
                     W2, b2.reshape(1, _NCLASS), bm=2000)


# speedup vs baseline: 3.5867x; 1.0052x over previous
import functools
import math

import jax
import jax.numpy as jnp
from jax import lax
from jax.experimental import pallas as pl
from jax.experimental.pallas import tpu as pltpu
from jax.experimental.pallas import tpu_sc as plsc

_NFEAT = 128
_NCLASS = 16
_B = 100000
_NC = 2
_NS = 16
_NW = _NC * _NS
_CHUNK = 128
_CPW = 25
_B_PAD = _NW * _CPW * _CHUNK
_NSLOT = 4
_DEPTH = 3


def _sc_gather(x, idx2d):
    mesh = plsc.VectorSubcoreMesh(core_axis_name="c", subcore_axis_name="s")

    @functools.partial(
        pl.kernel,
        out_type=jax.ShapeDtypeStruct((_B_PAD, _NFEAT), jnp.float32),
        mesh=mesh,
        scratch_types=[
            pltpu.VMEM((_CPW + 7, _CHUNK), jnp.int32),
            pltpu.VMEM((_NSLOT, _CHUNK, _NFEAT), jnp.float32),
            pltpu.SemaphoreType.DMA((_NSLOT,)),
            pltpu.SemaphoreType.DMA((_NSLOT,)),
        ],
    )
    def gather_kernel(x_hbm, idx_hbm, out_hbm, idx_v, rows_v, gsems, ssems):
        wid = lax.axis_index("s") * _NC + lax.axis_index("c")
        row0 = wid * _CPW
        start = pl.multiple_of(row0 // 8 * 8, 8)
        off = row0 - start
        pltpu.sync_copy(idx_hbm.at[pl.ds(start, _CPW + 7)], idx_v)

        def start_gather(c, slot):
            pltpu.async_copy(x_hbm.at[idx_v.at[off + c]], rows_v.at[slot],
                             gsems.at[slot])

        def scatter_desc(c, slot):
            return pltpu.make_async_copy(
                rows_v.at[slot],
                out_hbm.at[pl.ds((row0 + c) * _CHUNK, _CHUNK)],
                ssems.at[slot])

        for c in range(_DEPTH):
            start_gather(c, c)

        def body(j, carry):
            slot = lax.rem(j, _NSLOT)
            pltpu.make_async_copy(x_hbm.at[idx_v.at[off + j]],
                                  rows_v.at[slot], gsems.at[slot]).wait()
            scatter_desc(j, slot).start()
            n = j + _DEPTH
            nslot = lax.rem(n, _NSLOT)

            @pl.when(n < _CPW)
            def _():
                @pl.when(n >= _NSLOT)
                def _():
                    scatter_desc(n - _NSLOT, nslot).wait()
                start_gather(n, nslot)

            return carry

        lax.fori_loop(0, _CPW, body, 0)

        for c in range(_CPW - _NSLOT, _CPW):
            scatter_desc(c, c % _NSLOT).wait()

    return gather_kernel(x, idx2d)


def _tc_fused(anchor, hop_feat, W1, b1, W2, b2, bm):
    scale = 1.0 / math.sqrt(float(_NFEAT))

    def body(a_ref, h_ref, w1_ref, b1_ref, w2_ref, b2_ref, o_ref):
        a = a_ref[...]
        norm = jnp.sqrt(jnp.sum(a * a, axis=1, keepdims=True))
        an = a / jnp.maximum(norm, 1e-12)
        h0 = h_ref[0]
        h1 = h_ref[1]
        h2 = h_ref[2]
        l0 = jnp.sum(an * an, axis=1, keepdims=True) * scale
        l1 = jnp.sum(an * h0, axis=1, keepdims=True) * scale
        l2 = jnp.sum(an * h1, axis=1, keepdims=True) * scale
        l3 = jnp.sum(an * h2, axis=1, keepdims=True) * scale
        m = jnp.maximum(jnp.maximum(l0, l1), jnp.maximum(l2, l3))
        e0 = jnp.exp(l0 - m)
        e1 = jnp.exp(l1 - m)
        e2 = jnp.exp(l2 - m)
        e3 = jnp.exp(l3 - m)
        inv = 1.0 / (e0 + e1 + e2 + e3)
        pooled = (e0 * an + e1 * h0 + e2 * h1 + e3 * h2) * inv
        h = jnp.dot(pooled, w1_ref[...], preferred_element_type=jnp.float32)
        h = jnp.maximum(h + b1_ref[...], 0.0)
        o = jnp.dot(h, w2_ref[...], preferred_element_type=jnp.float32) + b2_ref[...]
        om = jnp.max(o, axis=1, keepdims=True)
        o_ref[...] = (o - om) - jnp.log(
            jnp.sum(jnp.exp(o - om), axis=1, keepdims=True))

    return pl.pallas_call(
        body,
        grid=(_B // bm,),
        in_specs=[
            pl.BlockSpec((bm, _NFEAT), lambda i: (i, 0)),
            pl.BlockSpec((3, bm, _NFEAT), lambda i: (0, i, 0)),
            pl.BlockSpec((_NFEAT, _NFEAT), lambda i: (0, 0)),
            pl.BlockSpec((1, _NFEAT), lambda i: (0, 0)),
            pl.BlockSpec((_NFEAT, _NCLASS), lambda i: (0, 0)),
            pl.BlockSpec((1, _NCLASS), lambda i: (0, 0)),
        ],
        out_specs=pl.BlockSpec((bm, _NCLASS), lambda i: (i, 0)),
        out_shape=jax.ShapeDtypeStruct((_B, _NCLASS), jnp.float32),
    )(anchor, hop_feat, W1, b1, W2, b2)


def kernel(x, hop_feat, idx, W1, b1, W2, b2):
    idx32 = idx.astype(jnp.int32)
    idx_pad = jnp.concatenate(
        [idx32, jnp.zeros((_B_PAD - _B,), jnp.int32)])
    idx2d = idx_pad.reshape(_NW * _CPW, _CHUNK)
    anchor = _sc_gather(x, idx2d)
    return _tc_fused(anchor, hop_feat, W1, b1.reshape(1, _NFEAT),
                     W2, b2.reshape(1, _NCLASS), bm=2000)
